# Initial kernel scaffold; baseline (speedup 1.0000x reference)
import jax
import jax.numpy as jnp
from jax.experimental import pallas as pl


def kernel(point, feat, row_splits, Wq, bq, Wk, bk, Wv, bv, Wp1, bp1, gp, betap, Wp2, bp2, gw1, betaw1, Ww1, bww1, gw2, betaw2, Ww2, bww2):
    raise NotImplementedError("write your pallas kernel here")


# speedup vs baseline: 3.2729x; 3.2729x over previous
import functools

import jax
import jax.numpy as jnp
from jax import lax
from jax.experimental import pallas as pl
from jax.experimental.pallas import tpu as pltpu
from jax.experimental.pallas import tpu_sc as plsc

N = 10000
NPAD = 10240
D = 128
NS = 16
SHARE = 8
MIDS = D // SHARE
EPS = 1e-5
TBL = 384

NW = 32
ROWS = N * NS
ROWS_PAD = 163840
RPW = ROWS_PAD // NW
CHUNK = 128
NCHUNK = RPW // CHUNK

_HI = jax.lax.Precision.HIGHEST


def _k1_body(feat_ref, point_ref, wq_ref, bq_ref, wk_ref, bk_ref, wv_ref,
             bv_ref, wp1_ref, table_ref, fq_ref, tq_ref):
    f = feat_ref[...]
    p = point_ref[...]
    fq_ref[...] = jnp.dot(f, wq_ref[...], precision=_HI) + bq_ref[...]
    table_ref[:, 0:D] = jnp.dot(f, wk_ref[...], precision=_HI) + bk_ref[...]
    table_ref[:, D:2 * D] = jnp.dot(f, wv_ref[...], precision=_HI) + bv_ref[...]
    t = jnp.dot(p, wp1_ref[...], precision=_HI)
    table_ref[:, 2 * D:TBL] = jnp.concatenate(
        [t, jnp.zeros((t.shape[0], TBL - 2 * D - 3), jnp.float32)], axis=1)
    tq_ref[...] = jnp.concatenate(
        [t, jnp.zeros((t.shape[0], NS - 3), jnp.float32)], axis=1)


def _project(feat, point, Wq, bq, Wk, bk, Wv, bv, Wp1):
    B = 400
    grid = N // B
    return pl.pallas_call(
        _k1_body,
        grid=(grid,),
        in_specs=[
            pl.BlockSpec((B, D), lambda i: (i, 0)),
            pl.BlockSpec((B, 3), lambda i: (i, 0)),
            pl.BlockSpec((D, D), lambda i: (0, 0)),
            pl.BlockSpec((D,), lambda i: (0,)),
            pl.BlockSpec((D, D), lambda i: (0, 0)),
            pl.BlockSpec((D,), lambda i: (0,)),
            pl.BlockSpec((D, D), lambda i: (0, 0)),
            pl.BlockSpec((D,), lambda i: (0,)),
            pl.BlockSpec((3, 3), lambda i: (0, 0)),
        ],
        out_specs=[
            pl.BlockSpec((B, TBL), lambda i: (i, 0)),
            pl.BlockSpec((B, D), lambda i: (i, 0)),
            pl.BlockSpec((B, NS), lambda i: (i, 0)),
        ],
        out_shape=[
            jax.ShapeDtypeStruct((N, TBL), jnp.float32),
            jax.ShapeDtypeStruct((N, D), jnp.float32),
            jax.ShapeDtypeStruct((N, NS), jnp.float32),
        ],
    )(feat, point, Wq, bq, Wk, bk, Wv, bv, Wp1)


def _k2_body(q_ref, kt_ref, idx_ref):
    q = q_ref[...]
    kt = kt_ref[...]
    qk = jnp.dot(q.astype(jnp.bfloat16), kt.astype(jnp.bfloat16),
                 preferred_element_type=jnp.float32)
    sqq = jnp.sum(q * q, axis=1, keepdims=True)
    sqk = jnp.sum(kt * kt, axis=0, keepdims=True)
    vals = sqq + sqk - 2.0 * qk
    bq = q.shape[0]
    col = lax.broadcasted_iota(jnp.int32, (bq, NPAD), 1)
    cols = []
    for _ in range(NS):
        m = jnp.min(vals, axis=1, keepdims=True)
        am = jnp.min(jnp.where(vals == m, col, jnp.int32(2**30)),
                     axis=1, keepdims=True)
        cols.append(am)
        vals = jnp.where(col == am, jnp.float32(3e38), vals)
    idx_ref[...] = jnp.concatenate(cols, axis=1)


def _knn(pts_pad, pts_t):
    BQ = 256
    grid = NPAD // BQ
    return pl.pallas_call(
        _k2_body,
        grid=(grid,),
        in_specs=[
            pl.BlockSpec((BQ, 8), lambda i: (i, 0)),
            pl.BlockSpec((8, NPAD), lambda i: (0, 0)),
        ],
        out_specs=pl.BlockSpec((BQ, NS), lambda i: (i, 0)),
        out_shape=jax.ShapeDtypeStruct((NPAD, NS), jnp.int32),
    )(pts_pad, pts_t)


def _sc_gather(table, idx_flat):
    mesh = plsc.VectorSubcoreMesh(core_axis_name="c", subcore_axis_name="s")

    @functools.partial(
        pl.kernel,
        mesh=mesh,
        out_type=jax.ShapeDtypeStruct((ROWS_PAD, TBL), jnp.float32),
        scratch_types=[
            pltpu.VMEM((CHUNK,), jnp.int32),
            pltpu.VMEM((CHUNK, TBL), jnp.float32),
            pltpu.SemaphoreType.DMA,
        ],
    )
    def gather_kernel(table_hbm, idx_hbm, out_hbm, idx_v, rows_v, sem):
        wid = lax.axis_index("s") * 2 + lax.axis_index("c")
        base = wid * RPW

        def chunk(c, carry):
            off = base + c * CHUNK
            pltpu.sync_copy(idx_hbm.at[pl.ds(off, CHUNK)], idx_v)
            pltpu.async_copy(table_hbm.at[idx_v], rows_v, sem).wait()
            pltpu.sync_copy(rows_v, out_hbm.at[pl.ds(off, CHUNK)])
            return carry

        lax.fori_loop(0, NCHUNK, chunk, 0)

    return gather_kernel(table, idx_flat)


def _k4_body(g_ref, q_ref, tq_ref, bp1_ref, gp_ref, bpp_ref, wp2_ref,
             bp2_ref, gw1_ref, bw1_ref, ww1_ref, bww1_ref, gw2_ref, bw2_ref,
             ww2_ref, bww2_ref, out_ref):
    B = q_ref.shape[0]
    BL = B * NS
    inv = jnp.float32(1.0 / jnp.sqrt(1.0 + EPS))

    g = g_ref[...]
    kg = g[:, 0:D]
    vg = g[:, D:2 * D]
    tn = g[:, 2 * D:2 * D + 3]

    ti = tq_ref[:, 0:3]
    pr3 = tn.reshape(B, NS, 3) - ti[:, None, :]
    pr3 = pr3 + bp1_ref[...]
    pr3 = jax.nn.relu(pr3 * inv * gp_ref[...] + bpp_ref[...])
    point_r = jnp.dot(pr3.reshape(BL, 3), wp2_ref[...],
                      precision=_HI) + bp2_ref[...]

    q = q_ref[...]
    qb = jnp.broadcast_to(q[:, None, :], (B, NS, D)).reshape(BL, D)
    w = kg - qb + point_r
    w = jax.nn.relu(w * inv * gw1_ref[...] + bw1_ref[...])
    w = jnp.dot(w, ww1_ref[...], precision=_HI) + bww1_ref[...]
    w = jax.nn.relu(w * inv * gw2_ref[...] + bw2_ref[...])
    w = jnp.dot(w, ww2_ref[...], precision=_HI) + bww2_ref[...]

    m = jnp.max(w, axis=1, keepdims=True)
    e = jnp.exp(w - m)
    w = e / jnp.sum(e, axis=1, keepdims=True)

    wt = jnp.concatenate([w] * SHARE, axis=1)
    fv = (vg + point_r) * wt
    out_ref[...] = jnp.sum(fv.reshape(B, NS, D), axis=1)


def _attn(gathered, feat_q, t_own, bp1, gp, betap, Wp2, bp2, gw1, betaw1,
          Ww1, bww1, gw2, betaw2, Ww2, bww2):
    B = 400
    grid = N // B
    full = lambda shape: pl.BlockSpec(shape, lambda i: tuple(0 for _ in shape))
    return pl.pallas_call(
        _k4_body,
        grid=(grid,),
        in_specs=[
            pl.BlockSpec((B * NS, TBL), lambda i: (i, 0)),
            pl.BlockSpec((B, D), lambda i: (i, 0)),
            pl.BlockSpec((B, NS), lambda i: (i, 0)),
            full((3,)), full((3,)), full((3,)),
            full((3, D)), full((D,)),
            full((D,)), full((D,)),
            full((D, MIDS)), full((MIDS,)),
            full((MIDS,)), full((MIDS,)),
            full((MIDS, MIDS)), full((MIDS,)),
        ],
        out_specs=pl.BlockSpec((B, D), lambda i: (i, 0)),
        out_shape=jax.ShapeDtypeStruct((N, D), jnp.float32),
    )(gathered, feat_q, t_own, bp1, gp, betap, Wp2, bp2, gw1, betaw1,
      Ww1, bww1, gw2, betaw2, Ww2, bww2)


def kernel(point, feat, row_splits, Wq, bq, Wk, bk, Wv, bv, Wp1, bp1, gp,
           betap, Wp2, bp2, gw1, betaw1, Ww1, bww1, gw2, betaw2, Ww2, bww2):
    table, feat_q, t_own = _project(feat, point, Wq, bq, Wk, bk, Wv, bv, Wp1)

    pts_pad = jnp.full((NPAD, 8), 1e4, jnp.float32)
    pts_pad = lax.dynamic_update_slice(
        pts_pad, jnp.pad(point, ((0, 0), (0, 5))), (0, 0))
    idx = _knn(pts_pad, pts_pad.T)[:N]

    idx_flat = jnp.pad(idx.reshape(-1), (0, ROWS_PAD - ROWS))
    gathered = _sc_gather(table, idx_flat)[:ROWS]

    return _attn(gathered, feat_q, t_own, bp1, gp, betap, Wp2, bp2,
                 gw1, betaw1, Ww1, bww1, gw2, betaw2, Ww2, bww2)


# speedup vs baseline: 6.1899x; 1.8912x over previous
import functools

import jax
import jax.numpy as jnp
from jax import lax
from jax.experimental import pallas as pl
from jax.experimental.pallas import tpu as pltpu
from jax.experimental.pallas import tpu_sc as plsc

N = 10000
NPAD = 10240
D = 128
NS = 16
SHARE = 8
MIDS = D // SHARE
EPS = 1e-5
TBL = 384

NW = 32
ROWS = N * NS
ROWS_PAD = 163840
RPW = ROWS_PAD // NW
CHUNK = 128
NCHUNK = RPW // CHUNK

_HI = jax.lax.Precision.HIGHEST


def _k1_body(feat_ref, point_ref, wq_ref, bq_ref, wk_ref, bk_ref, wv_ref,
             bv_ref, wp1_ref, table_ref, fq_ref, tq_ref):
    f = feat_ref[...]
    p = point_ref[...]
    fq_ref[...] = jnp.dot(f, wq_ref[...], precision=_HI) + bq_ref[...]
    table_ref[:, 0:D] = jnp.dot(f, wk_ref[...], precision=_HI) + bk_ref[...]
    table_ref[:, D:2 * D] = jnp.dot(f, wv_ref[...], precision=_HI) + bv_ref[...]
    t = jnp.dot(p, wp1_ref[...], precision=_HI)
    table_ref[:, 2 * D:TBL] = jnp.concatenate(
        [t, jnp.zeros((t.shape[0], TBL - 2 * D - 3), jnp.float32)], axis=1)
    tq_ref[...] = jnp.concatenate(
        [t, jnp.zeros((t.shape[0], NS - 3), jnp.float32)], axis=1)


def _project(feat, point, Wq, bq, Wk, bk, Wv, bv, Wp1):
    B = 400
    grid = N // B
    return pl.pallas_call(
        _k1_body,
        grid=(grid,),
        in_specs=[
            pl.BlockSpec((B, D), lambda i: (i, 0)),
            pl.BlockSpec((B, 3), lambda i: (i, 0)),
            pl.BlockSpec((D, D), lambda i: (0, 0)),
            pl.BlockSpec((D,), lambda i: (0,)),
            pl.BlockSpec((D, D), lambda i: (0, 0)),
            pl.BlockSpec((D,), lambda i: (0,)),
            pl.BlockSpec((D, D), lambda i: (0, 0)),
            pl.BlockSpec((D,), lambda i: (0,)),
            pl.BlockSpec((3, 3), lambda i: (0, 0)),
        ],
        out_specs=[
            pl.BlockSpec((B, TBL), lambda i: (i, 0)),
            pl.BlockSpec((B, D), lambda i: (i, 0)),
            pl.BlockSpec((B, NS), lambda i: (i, 0)),
        ],
        out_shape=[
            jax.ShapeDtypeStruct((N, TBL), jnp.float32),
            jax.ShapeDtypeStruct((N, D), jnp.float32),
            jax.ShapeDtypeStruct((N, NS), jnp.float32),
        ],
    )(feat, point, Wq, bq, Wk, bk, Wv, bv, Wp1)


def _k2_body(q_ref, kt_ref, idx_ref):
    q = q_ref[...]
    kt = kt_ref[...]
    qk = jnp.dot(q.astype(jnp.bfloat16), kt.astype(jnp.bfloat16),
                 preferred_element_type=jnp.float32)
    sqq = jnp.sum(q * q, axis=1, keepdims=True)
    sqk = jnp.sum(kt * kt, axis=0, keepdims=True)
    vals = sqq + sqk - 2.0 * qk
    bq = q.shape[0]

    W = 256
    G = NPAD // W
    BIGF = jnp.float32(3e38)
    BIGI = jnp.int32(2**30)
    m1 = jnp.full((bq, W), BIGF, jnp.float32)
    m2 = jnp.full((bq, W), BIGF, jnp.float32)
    m3 = jnp.full((bq, W), BIGF, jnp.float32)
    zi = jnp.zeros((bq, W), jnp.int32)
    a1, a2, a3 = zi, zi, zi
    for g in range(G):
        x = vals[:, g * W:(g + 1) * W]
        gi = jnp.int32(g)
        lt1 = x < m1
        lt2 = x < m2
        lt3 = x < m3
        a3 = jnp.where(lt2, a2, jnp.where(lt3, gi, a3))
        m3 = jnp.where(lt2, m2, jnp.where(lt3, x, m3))
        a2 = jnp.where(lt1, a1, jnp.where(lt2, gi, a2))
        m2 = jnp.where(lt1, m1, jnp.where(lt2, x, m2))
        a1 = jnp.where(lt1, gi, a1)
        m1 = jnp.where(lt1, x, m1)

    lane = lax.broadcasted_iota(jnp.int32, (bq, W), 1)
    cols = []
    for _ in range(NS):
        m = jnp.min(m1, axis=1, keepdims=True)
        cand = jnp.where(m1 == m, a1 * W + lane, BIGI)
        col = jnp.min(cand, axis=1, keepdims=True)
        cols.append(col)
        sel = lane == (col & (W - 1))
        m1 = jnp.where(sel, m2, m1)
        a1 = jnp.where(sel, a2, a1)
        m2 = jnp.where(sel, m3, m2)
        a2 = jnp.where(sel, a3, a2)
        m3 = jnp.where(sel, BIGF, m3)
    idx_ref[...] = jnp.concatenate(cols, axis=1)


def _knn(pts_pad, pts_t):
    BQ = 256
    grid = NPAD // BQ
    return pl.pallas_call(
        _k2_body,
        grid=(grid,),
        in_specs=[
            pl.BlockSpec((BQ, 8), lambda i: (i, 0)),
            pl.BlockSpec((8, NPAD), lambda i: (0, 0)),
        ],
        out_specs=pl.BlockSpec((BQ, NS), lambda i: (i, 0)),
        out_shape=jax.ShapeDtypeStruct((NPAD, NS), jnp.int32),
    )(pts_pad, pts_t)


def _sc_gather(table, idx_flat):
    mesh = plsc.VectorSubcoreMesh(core_axis_name="c", subcore_axis_name="s")

    @functools.partial(
        pl.kernel,
        mesh=mesh,
        out_type=jax.ShapeDtypeStruct((ROWS_PAD, TBL), jnp.float32),
        scratch_types=[
            pltpu.VMEM((CHUNK,), jnp.int32),
            pltpu.VMEM((CHUNK, TBL), jnp.float32),
            pltpu.SemaphoreType.DMA,
        ],
    )
    def gather_kernel(table_hbm, idx_hbm, out_hbm, idx_v, rows_v, sem):
        wid = lax.axis_index("s") * 2 + lax.axis_index("c")
        base = wid * RPW

        def chunk(c, carry):
            off = base + c * CHUNK
            pltpu.sync_copy(idx_hbm.at[pl.ds(off, CHUNK)], idx_v)
            pltpu.async_copy(table_hbm.at[idx_v], rows_v, sem).wait()
            pltpu.sync_copy(rows_v, out_hbm.at[pl.ds(off, CHUNK)])
            return carry

        lax.fori_loop(0, NCHUNK, chunk, 0)

    return gather_kernel(table, idx_flat)


def _k4_body(g_ref, q_ref, tq_ref, bp1_ref, gp_ref, bpp_ref, wp2_ref,
             bp2_ref, gw1_ref, bw1_ref, ww1_ref, bww1_ref, gw2_ref, bw2_ref,
             ww2_ref, bww2_ref, out_ref):
    B = q_ref.shape[0]
    BL = B * NS
    inv = jnp.float32(1.0 / jnp.sqrt(1.0 + EPS))

    g = g_ref[...]
    kg = g[:, 0:D]
    vg = g[:, D:2 * D]
    tn = g[:, 2 * D:2 * D + 3]

    ti = tq_ref[:, 0:3]
    pr3 = tn.reshape(B, NS, 3) - ti[:, None, :]
    pr3 = pr3 + bp1_ref[...]
    pr3 = jax.nn.relu(pr3 * inv * gp_ref[...] + bpp_ref[...])
    point_r = jnp.dot(pr3.reshape(BL, 3), wp2_ref[...],
                      precision=_HI) + bp2_ref[...]

    q = q_ref[...]
    qb = jnp.broadcast_to(q[:, None, :], (B, NS, D)).reshape(BL, D)
    w = kg - qb + point_r
    w = jax.nn.relu(w * inv * gw1_ref[...] + bw1_ref[...])
    w = jnp.dot(w, ww1_ref[...], precision=_HI) + bww1_ref[...]
    w = jax.nn.relu(w * inv * gw2_ref[...] + bw2_ref[...])
    w = jnp.dot(w, ww2_ref[...], precision=_HI) + bww2_ref[...]

    m = jnp.max(w, axis=1, keepdims=True)
    e = jnp.exp(w - m)
    w = e / jnp.sum(e, axis=1, keepdims=True)

    wt = jnp.concatenate([w] * SHARE, axis=1)
    fv = (vg + point_r) * wt
    out_ref[...] = jnp.sum(fv.reshape(B, NS, D), axis=1)


def _attn(gathered, feat_q, t_own, bp1, gp, betap, Wp2, bp2, gw1, betaw1,
          Ww1, bww1, gw2, betaw2, Ww2, bww2):
    B = 400
    grid = N // B
    full = lambda shape: pl.BlockSpec(shape, lambda i: tuple(0 for _ in shape))
    return pl.pallas_call(
        _k4_body,
        grid=(grid,),
        in_specs=[
            pl.BlockSpec((B * NS, TBL), lambda i: (i, 0)),
            pl.BlockSpec((B, D), lambda i: (i, 0)),
            pl.BlockSpec((B, NS), lambda i: (i, 0)),
            full((3,)), full((3,)), full((3,)),
            full((3, D)), full((D,)),
            full((D,)), full((D,)),
            full((D, MIDS)), full((MIDS,)),
            full((MIDS,)), full((MIDS,)),
            full((MIDS, MIDS)), full((MIDS,)),
        ],
        out_specs=pl.BlockSpec((B, D), lambda i: (i, 0)),
        out_shape=jax.ShapeDtypeStruct((N, D), jnp.float32),
    )(gathered, feat_q, t_own, bp1, gp, betap, Wp2, bp2, gw1, betaw1,
      Ww1, bww1, gw2, betaw2, Ww2, bww2)


def kernel(point, feat, row_splits, Wq, bq, Wk, bk, Wv, bv, Wp1, bp1, gp,
           betap, Wp2, bp2, gw1, betaw1, Ww1, bww1, gw2, betaw2, Ww2, bww2):
    table, feat_q, t_own = _project(feat, point, Wq, bq, Wk, bk, Wv, bv, Wp1)

    pts_pad = jnp.full((NPAD, 8), 1e4, jnp.float32)
    pts_pad = lax.dynamic_update_slice(
        pts_pad, jnp.pad(point, ((0, 0), (0, 5))), (0, 0))
    idx = _knn(pts_pad, pts_pad.T)[:N]

    idx_flat = jnp.pad(idx.reshape(-1), (0, ROWS_PAD - ROWS))
    gathered = _sc_gather(table, idx_flat)[:ROWS]

    return _attn(gathered, feat_q, t_own, bp1, gp, betap, Wp2, bp2,
                 gw1, betaw1, Ww1, bww1, gw2, betaw2, Ww2, bww2)


# speedup vs baseline: 6.3967x; 1.0334x over previous
import functools

import jax
import jax.numpy as jnp
from jax import lax
from jax.experimental import pallas as pl
from jax.experimental.pallas import tpu as pltpu
from jax.experimental.pallas import tpu_sc as plsc

N = 10000
NPAD = 10240
D = 128
NS = 16
SHARE = 8
MIDS = D // SHARE
EPS = 1e-5
TBL = 384

NW = 32
ROWS = N * NS
ROWS_PAD = 163840
RPW = ROWS_PAD // NW
CHUNK = 128
NCHUNK = RPW // CHUNK

_HI = jax.lax.Precision.HIGHEST


def _k1_body(feat_ref, point_ref, wq_ref, bq_ref, wk_ref, bk_ref, wv_ref,
             bv_ref, wp1_ref, table_ref, fq_ref, tq_ref):
    f = feat_ref[...]
    p = point_ref[...]
    fq_ref[...] = jnp.dot(f, wq_ref[...], precision=_HI) + bq_ref[...]
    table_ref[:, 0:D] = jnp.dot(f, wk_ref[...], precision=_HI) + bk_ref[...]
    table_ref[:, D:2 * D] = jnp.dot(f, wv_ref[...], precision=_HI) + bv_ref[...]
    t = jnp.dot(p, wp1_ref[...], precision=_HI)
    table_ref[:, 2 * D:TBL] = jnp.concatenate(
        [t, jnp.zeros((t.shape[0], TBL - 2 * D - 3), jnp.float32)], axis=1)
    tq_ref[...] = jnp.concatenate(
        [t, jnp.zeros((t.shape[0], NS - 3), jnp.float32)], axis=1)


def _project(feat, point, Wq, bq, Wk, bk, Wv, bv, Wp1):
    B = 400
    grid = N // B
    return pl.pallas_call(
        _k1_body,
        grid=(grid,),
        in_specs=[
            pl.BlockSpec((B, D), lambda i: (i, 0)),
            pl.BlockSpec((B, 3), lambda i: (i, 0)),
            pl.BlockSpec((D, D), lambda i: (0, 0)),
            pl.BlockSpec((D,), lambda i: (0,)),
            pl.BlockSpec((D, D), lambda i: (0, 0)),
            pl.BlockSpec((D,), lambda i: (0,)),
            pl.BlockSpec((D, D), lambda i: (0, 0)),
            pl.BlockSpec((D,), lambda i: (0,)),
            pl.BlockSpec((3, 3), lambda i: (0, 0)),
        ],
        out_specs=[
            pl.BlockSpec((B, TBL), lambda i: (i, 0)),
            pl.BlockSpec((B, D), lambda i: (i, 0)),
            pl.BlockSpec((B, NS), lambda i: (i, 0)),
        ],
        out_shape=[
            jax.ShapeDtypeStruct((N, TBL), jnp.float32),
            jax.ShapeDtypeStruct((N, D), jnp.float32),
            jax.ShapeDtypeStruct((N, NS), jnp.float32),
        ],
    )(feat, point, Wq, bq, Wk, bk, Wv, bv, Wp1)


def _k2_body(q_ref, kt_ref, idx_ref):
    q = q_ref[...]
    kt = kt_ref[...]
    qk = jnp.dot(q.astype(jnp.bfloat16), kt.astype(jnp.bfloat16),
                 preferred_element_type=jnp.float32)
    sqq = jnp.sum(q * q, axis=1, keepdims=True)
    sqk = jnp.sum(kt * kt, axis=0, keepdims=True)
    vals = sqq + sqk - 2.0 * qk
    bq = q.shape[0]

    W = 256
    G = NPAD // W
    BIGF = jnp.float32(3e38)
    BIGI = jnp.int32(2**30)
    m1 = jnp.full((bq, W), BIGF, jnp.float32)
    m2 = jnp.full((bq, W), BIGF, jnp.float32)
    m3 = jnp.full((bq, W), BIGF, jnp.float32)
    zi = jnp.zeros((bq, W), jnp.int32)
    a1, a2, a3 = zi, zi, zi
    for g in range(G):
        x = vals[:, g * W:(g + 1) * W]
        gi = jnp.int32(g)
        lt1 = x < m1
        lt2 = x < m2
        lt3 = x < m3
        a3 = jnp.where(lt2, a2, jnp.where(lt3, gi, a3))
        m3 = jnp.where(lt2, m2, jnp.where(lt3, x, m3))
        a2 = jnp.where(lt1, a1, jnp.where(lt2, gi, a2))
        m2 = jnp.where(lt1, m1, jnp.where(lt2, x, m2))
        a1 = jnp.where(lt1, gi, a1)
        m1 = jnp.where(lt1, x, m1)

    lane = lax.broadcasted_iota(jnp.int32, (bq, W), 1)
    cols = []
    for _ in range(NS):
        m = jnp.min(m1, axis=1, keepdims=True)
        cand = jnp.where(m1 == m, a1 * W + lane, BIGI)
        col = jnp.min(cand, axis=1, keepdims=True)
        cols.append(col)
        sel = lane == (col & (W - 1))
        m1 = jnp.where(sel, m2, m1)
        a1 = jnp.where(sel, a2, a1)
        m2 = jnp.where(sel, m3, m2)
        a2 = jnp.where(sel, a3, a2)
        m3 = jnp.where(sel, BIGF, m3)
    idx_ref[...] = jnp.concatenate(cols, axis=1)


def _knn(pts_pad, pts_t):
    BQ = 256
    grid = NPAD // BQ
    return pl.pallas_call(
        _k2_body,
        grid=(grid,),
        in_specs=[
            pl.BlockSpec((BQ, 8), lambda i: (i, 0)),
            pl.BlockSpec((8, NPAD), lambda i: (0, 0)),
        ],
        out_specs=pl.BlockSpec((BQ, NS), lambda i: (i, 0)),
        out_shape=jax.ShapeDtypeStruct((NPAD, NS), jnp.int32),
    )(pts_pad, pts_t)


def _sc_gather(table, idx_flat):
    mesh = plsc.VectorSubcoreMesh(core_axis_name="c", subcore_axis_name="s")

    @functools.partial(
        pl.kernel,
        mesh=mesh,
        out_type=jax.ShapeDtypeStruct((ROWS_PAD, TBL), jnp.float32),
        scratch_types=[
            pltpu.VMEM((RPW,), jnp.int32),
            pltpu.VMEM((CHUNK, TBL), jnp.float32),
            pltpu.VMEM((CHUNK, TBL), jnp.float32),
            pltpu.SemaphoreType.DMA,
            pltpu.SemaphoreType.DMA,
            pltpu.SemaphoreType.DMA,
            pltpu.SemaphoreType.DMA,
        ],
    )
    def gather_kernel(table_hbm, idx_hbm, out_hbm, idx_v, buf0, buf1,
                      sg0, sg1, ss0, ss1):
        wid = lax.axis_index("s") * 2 + lax.axis_index("c")
        base = wid * RPW
        pltpu.sync_copy(idx_hbm.at[pl.ds(base, RPW)], idx_v)

        def pair(i, carry):
            c0 = 2 * i
            c1 = 2 * i + 1
            o0 = base + c0 * CHUNK
            o1 = base + c1 * CHUNK

            @pl.when(i > 0)
            def _():
                pltpu.make_async_copy(
                    buf0, out_hbm.at[pl.ds(o0, CHUNK)], ss0).wait()

            g0 = pltpu.async_copy(
                table_hbm.at[idx_v.at[pl.ds(c0 * CHUNK, CHUNK)]], buf0, sg0)

            @pl.when(i > 0)
            def _():
                pltpu.make_async_copy(
                    buf1, out_hbm.at[pl.ds(o1, CHUNK)], ss1).wait()

            g1 = pltpu.async_copy(
                table_hbm.at[idx_v.at[pl.ds(c1 * CHUNK, CHUNK)]], buf1, sg1)
            g0.wait()
            pltpu.async_copy(buf0, out_hbm.at[pl.ds(o0, CHUNK)], ss0)
            g1.wait()
            pltpu.async_copy(buf1, out_hbm.at[pl.ds(o1, CHUNK)], ss1)
            return carry

        lax.fori_loop(0, NCHUNK // 2, pair, 0)
        pltpu.make_async_copy(buf0, out_hbm.at[pl.ds(base, CHUNK)], ss0).wait()
        pltpu.make_async_copy(buf1, out_hbm.at[pl.ds(base, CHUNK)], ss1).wait()

    return gather_kernel(table, idx_flat)


def _k4_body(g_ref, q_ref, tq_ref, bp1_ref, gp_ref, bpp_ref, wp2_ref,
             bp2_ref, gw1_ref, bw1_ref, ww1_ref, bww1_ref, gw2_ref, bw2_ref,
             ww2_ref, bww2_ref, out_ref):
    B = q_ref.shape[0]
    BL = B * NS
    inv = jnp.float32(1.0 / jnp.sqrt(1.0 + EPS))

    g = g_ref[...]
    kg = g[:, 0:D]
    vg = g[:, D:2 * D]
    tn = g[:, 2 * D:2 * D + 3]

    ti = tq_ref[:, 0:3]
    pr3 = tn.reshape(B, NS, 3) - ti[:, None, :]
    pr3 = pr3 + bp1_ref[...]
    pr3 = jax.nn.relu(pr3 * inv * gp_ref[...] + bpp_ref[...])
    point_r = jnp.dot(pr3.reshape(BL, 3), wp2_ref[...],
                      precision=_HI) + bp2_ref[...]

    q = q_ref[...]
    qb = jnp.broadcast_to(q[:, None, :], (B, NS, D)).reshape(BL, D)
    w = kg - qb + point_r
    w = jax.nn.relu(w * inv * gw1_ref[...] + bw1_ref[...])
    w = jnp.dot(w, ww1_ref[...], precision=_HI) + bww1_ref[...]
    w = jax.nn.relu(w * inv * gw2_ref[...] + bw2_ref[...])
    w = jnp.dot(w, ww2_ref[...], precision=_HI) + bww2_ref[...]

    m = jnp.max(w, axis=1, keepdims=True)
    e = jnp.exp(w - m)
    w = e / jnp.sum(e, axis=1, keepdims=True)

    wt = jnp.concatenate([w] * SHARE, axis=1)
    fv = (vg + point_r) * wt
    out_ref[...] = jnp.sum(fv.reshape(B, NS, D), axis=1)


def _attn(gathered, feat_q, t_own, bp1, gp, betap, Wp2, bp2, gw1, betaw1,
          Ww1, bww1, gw2, betaw2, Ww2, bww2):
    B = 400
    grid = N // B
    full = lambda shape: pl.BlockSpec(shape, lambda i: tuple(0 for _ in shape))
    return pl.pallas_call(
        _k4_body,
        grid=(grid,),
        in_specs=[
            pl.BlockSpec((B * NS, TBL), lambda i: (i, 0)),
            pl.BlockSpec((B, D), lambda i: (i, 0)),
            pl.BlockSpec((B, NS), lambda i: (i, 0)),
            full((3,)), full((3,)), full((3,)),
            full((3, D)), full((D,)),
            full((D,)), full((D,)),
            full((D, MIDS)), full((MIDS,)),
            full((MIDS,)), full((MIDS,)),
            full((MIDS, MIDS)), full((MIDS,)),
        ],
        out_specs=pl.BlockSpec((B, D), lambda i: (i, 0)),
        out_shape=jax.ShapeDtypeStruct((N, D), jnp.float32),
    )(gathered, feat_q, t_own, bp1, gp, betap, Wp2, bp2, gw1, betaw1,
      Ww1, bww1, gw2, betaw2, Ww2, bww2)


def kernel(point, feat, row_splits, Wq, bq, Wk, bk, Wv, bv, Wp1, bp1, gp,
           betap, Wp2, bp2, gw1, betaw1, Ww1, bww1, gw2, betaw2, Ww2, bww2):
    table, feat_q, t_own = _project(feat, point, Wq, bq, Wk, bk, Wv, bv, Wp1)

    pts_pad = jnp.full((NPAD, 8), 1e4, jnp.float32)
    pts_pad = lax.dynamic_update_slice(
        pts_pad, jnp.pad(point, ((0, 0), (0, 5))), (0, 0))
    idx = _knn(pts_pad, pts_pad.T)[:N]

    idx_flat = jnp.pad(idx.reshape(-1), (0, ROWS_PAD - ROWS))
    gathered = _sc_gather(table, idx_flat)[:ROWS]

    return _attn(gathered, feat_q, t_own, bp1, gp, betap, Wp2, bp2,
                 gw1, betaw1, Ww1, bww1, gw2, betaw2, Ww2, bww2)


# speedup vs baseline: 7.6911x; 1.2023x over previous
import functools

import jax
import jax.numpy as jnp
from jax import lax
from jax.experimental import pallas as pl
from jax.experimental.pallas import tpu as pltpu
from jax.experimental.pallas import tpu_sc as plsc

N = 10000
NPAD = 10240
D = 128
NS = 16
SHARE = 8
MIDS = D // SHARE
EPS = 1e-5
TBL = 384

NH = 5000
NPAD_H = 5120

NW = 32
ROWS = NH * NS
ROWS_PAD = 81920
RPW = ROWS_PAD // NW
CHUNK = 128
NCHUNK = RPW // CHUNK

_HI = jax.lax.Precision.HIGHEST


def _k1_body(feat_ref, point_ref, wq_ref, bq_ref, wk_ref, bk_ref, wv_ref,
             bv_ref, wp1_ref, table_ref, fq_ref, tq_ref):
    f = feat_ref[...]
    p = point_ref[...]
    fq_ref[...] = jnp.dot(f, wq_ref[...], precision=_HI) + bq_ref[...]
    table_ref[:, 0:D] = jnp.dot(f, wk_ref[...], precision=_HI) + bk_ref[...]
    table_ref[:, D:2 * D] = jnp.dot(f, wv_ref[...], precision=_HI) + bv_ref[...]
    t = jnp.dot(p, wp1_ref[...], precision=_HI)
    table_ref[:, 2 * D:TBL] = jnp.concatenate(
        [t, jnp.zeros((t.shape[0], TBL - 2 * D - 3), jnp.float32)], axis=1)
    tq_ref[...] = jnp.concatenate(
        [t, jnp.zeros((t.shape[0], NS - 3), jnp.float32)], axis=1)


def _project(feat, point, Wq, bq, Wk, bk, Wv, bv, Wp1):
    B = 400
    grid = N // B
    return pl.pallas_call(
        _k1_body,
        grid=(grid,),
        in_specs=[
            pl.BlockSpec((B, D), lambda i: (i, 0)),
            pl.BlockSpec((B, 3), lambda i: (i, 0)),
            pl.BlockSpec((D, D), lambda i: (0, 0)),
            pl.BlockSpec((D,), lambda i: (0,)),
            pl.BlockSpec((D, D), lambda i: (0, 0)),
            pl.BlockSpec((D,), lambda i: (0,)),
            pl.BlockSpec((D, D), lambda i: (0, 0)),
            pl.BlockSpec((D,), lambda i: (0,)),
            pl.BlockSpec((3, 3), lambda i: (0, 0)),
        ],
        out_specs=[
            pl.BlockSpec((B, TBL), lambda i: (i, 0)),
            pl.BlockSpec((B, D), lambda i: (i, 0)),
            pl.BlockSpec((B, NS), lambda i: (i, 0)),
        ],
        out_shape=[
            jax.ShapeDtypeStruct((N, TBL), jnp.float32),
            jax.ShapeDtypeStruct((N, D), jnp.float32),
            jax.ShapeDtypeStruct((N, NS), jnp.float32),
        ],
    )(feat, point, Wq, bq, Wk, bk, Wv, bv, Wp1)


def _k2_body(q_ref, kt_ref, idx_ref):
    q = q_ref[...]
    kt = kt_ref[...]
    qk = jnp.dot(q.astype(jnp.bfloat16), kt.astype(jnp.bfloat16),
                 preferred_element_type=jnp.float32)
    sqq = jnp.sum(q * q, axis=1, keepdims=True)
    sqk = jnp.sum(kt * kt, axis=0, keepdims=True)
    vals = sqq + sqk - 2.0 * qk
    bq = q.shape[0]

    W = 256
    G = NPAD // W
    BIGF = jnp.float32(3e38)
    BIGI = jnp.int32(2**30)
    m1 = jnp.full((bq, W), BIGF, jnp.float32)
    m2 = jnp.full((bq, W), BIGF, jnp.float32)
    m3 = jnp.full((bq, W), BIGF, jnp.float32)
    zi = jnp.zeros((bq, W), jnp.int32)
    a1, a2, a3 = zi, zi, zi
    for g in range(G):
        x = vals[:, g * W:(g + 1) * W]
        gi = jnp.int32(g)
        lt1 = x < m1
        lt2 = x < m2
        lt3 = x < m3
        a3 = jnp.where(lt2, a2, jnp.where(lt3, gi, a3))
        m3 = jnp.where(lt2, m2, jnp.where(lt3, x, m3))
        a2 = jnp.where(lt1, a1, jnp.where(lt2, gi, a2))
        m2 = jnp.where(lt1, m1, jnp.where(lt2, x, m2))
        a1 = jnp.where(lt1, gi, a1)
        m1 = jnp.where(lt1, x, m1)

    lane = lax.broadcasted_iota(jnp.int32, (bq, W), 1)
    cols = []
    for _ in range(NS):
        m = jnp.min(m1, axis=1, keepdims=True)
        cand = jnp.where(m1 == m, a1 * W + lane, BIGI)
        col = jnp.min(cand, axis=1, keepdims=True)
        cols.append(col)
        sel = lane == (col & (W - 1))
        m1 = jnp.where(sel, m2, m1)
        a1 = jnp.where(sel, a2, a1)
        m2 = jnp.where(sel, m3, m2)
        a2 = jnp.where(sel, a3, a2)
        m3 = jnp.where(sel, BIGF, m3)
    idx_ref[...] = jnp.concatenate(cols, axis=1)


def _knn(pts_pad_q, pts_t):
    BQ = 256
    nq = pts_pad_q.shape[0]
    grid = nq // BQ
    return pl.pallas_call(
        _k2_body,
        grid=(grid,),
        in_specs=[
            pl.BlockSpec((BQ, 8), lambda i: (i, 0)),
            pl.BlockSpec((8, NPAD), lambda i: (0, 0)),
        ],
        out_specs=pl.BlockSpec((BQ, NS), lambda i: (i, 0)),
        out_shape=jax.ShapeDtypeStruct((nq, NS), jnp.int32),
    )(pts_pad_q, pts_t)


def _sc_gather(table, idx_flat):
    mesh = plsc.VectorSubcoreMesh(core_axis_name="c", subcore_axis_name="s")

    @functools.partial(
        pl.kernel,
        mesh=mesh,
        out_type=jax.ShapeDtypeStruct((ROWS_PAD, TBL), jnp.float32),
        scratch_types=[
            pltpu.VMEM((RPW,), jnp.int32),
            pltpu.VMEM((CHUNK, TBL), jnp.float32),
            pltpu.VMEM((CHUNK, TBL), jnp.float32),
            pltpu.SemaphoreType.DMA,
            pltpu.SemaphoreType.DMA,
            pltpu.SemaphoreType.DMA,
            pltpu.SemaphoreType.DMA,
        ],
    )
    def gather_kernel(table_hbm, idx_hbm, out_hbm, idx_v, buf0, buf1,
                      sg0, sg1, ss0, ss1):
        wid = lax.axis_index("s") * 2 + lax.axis_index("c")
        base = wid * RPW
        pltpu.sync_copy(idx_hbm.at[pl.ds(base, RPW)], idx_v)

        def pair(i, carry):
            c0 = 2 * i
            c1 = 2 * i + 1
            o0 = base + c0 * CHUNK
            o1 = base + c1 * CHUNK

            @pl.when(i > 0)
            def _():
                pltpu.make_async_copy(
                    buf0, out_hbm.at[pl.ds(o0, CHUNK)], ss0).wait()

            g0 = pltpu.async_copy(
                table_hbm.at[idx_v.at[pl.ds(c0 * CHUNK, CHUNK)]], buf0, sg0)

            @pl.when(i > 0)
            def _():
                pltpu.make_async_copy(
                    buf1, out_hbm.at[pl.ds(o1, CHUNK)], ss1).wait()

            g1 = pltpu.async_copy(
                table_hbm.at[idx_v.at[pl.ds(c1 * CHUNK, CHUNK)]], buf1, sg1)
            g0.wait()
            pltpu.async_copy(buf0, out_hbm.at[pl.ds(o0, CHUNK)], ss0)
            g1.wait()
            pltpu.async_copy(buf1, out_hbm.at[pl.ds(o1, CHUNK)], ss1)
            return carry

        lax.fori_loop(0, NCHUNK // 2, pair, 0)
        pltpu.make_async_copy(buf0, out_hbm.at[pl.ds(base, CHUNK)], ss0).wait()
        pltpu.make_async_copy(buf1, out_hbm.at[pl.ds(base, CHUNK)], ss1).wait()

    return gather_kernel(table, idx_flat)


def _k4_body(g_ref, q_ref, tq_ref, bp1_ref, gp_ref, bpp_ref, wp2_ref,
             bp2_ref, gw1_ref, bw1_ref, ww1_ref, bww1_ref, gw2_ref, bw2_ref,
             ww2_ref, bww2_ref, out_ref):
    B = q_ref.shape[0]
    BL = B * NS
    inv = jnp.float32(1.0 / jnp.sqrt(1.0 + EPS))

    g = g_ref[...]
    kg = g[:, 0:D]
    vg = g[:, D:2 * D]
    tn = g[:, 2 * D:2 * D + 3]

    ti = tq_ref[:, 0:3]
    pr3 = tn.reshape(B, NS, 3) - ti[:, None, :]
    pr3 = pr3 + bp1_ref[...]
    pr3 = jax.nn.relu(pr3 * inv * gp_ref[...] + bpp_ref[...])
    point_r = jnp.dot(pr3.reshape(BL, 3), wp2_ref[...],
                      precision=_HI) + bp2_ref[...]

    q = q_ref[...]
    qb = jnp.broadcast_to(q[:, None, :], (B, NS, D)).reshape(BL, D)
    w = kg - qb + point_r
    w = jax.nn.relu(w * inv * gw1_ref[...] + bw1_ref[...])
    w = jnp.dot(w, ww1_ref[...], precision=_HI) + bww1_ref[...]
    w = jax.nn.relu(w * inv * gw2_ref[...] + bw2_ref[...])
    w = jnp.dot(w, ww2_ref[...], precision=_HI) + bww2_ref[...]

    m = jnp.max(w, axis=1, keepdims=True)
    e = jnp.exp(w - m)
    w = e / jnp.sum(e, axis=1, keepdims=True)

    wt = jnp.concatenate([w] * SHARE, axis=1)
    fv = (vg + point_r) * wt
    out_ref[...] = jnp.sum(fv.reshape(B, NS, D), axis=1)


def _attn(gathered, feat_q, t_own, bp1, gp, betap, Wp2, bp2, gw1, betaw1,
          Ww1, bww1, gw2, betaw2, Ww2, bww2):
    B = 200
    grid = NH // B
    full = lambda shape: pl.BlockSpec(shape, lambda i: tuple(0 for _ in shape))
    return pl.pallas_call(
        _k4_body,
        grid=(grid,),
        in_specs=[
            pl.BlockSpec((B * NS, TBL), lambda i: (i, 0)),
            pl.BlockSpec((B, D), lambda i: (i, 0)),
            pl.BlockSpec((B, NS), lambda i: (i, 0)),
            full((3,)), full((3,)), full((3,)),
            full((3, D)), full((D,)),
            full((D,)), full((D,)),
            full((D, MIDS)), full((MIDS,)),
            full((MIDS,)), full((MIDS,)),
            full((MIDS, MIDS)), full((MIDS,)),
        ],
        out_specs=pl.BlockSpec((B, D), lambda i: (i, 0)),
        out_shape=jax.ShapeDtypeStruct((NH, D), jnp.float32),
    )(gathered, feat_q, t_own, bp1, gp, betap, Wp2, bp2, gw1, betaw1,
      Ww1, bww1, gw2, betaw2, Ww2, bww2)


def kernel(point, feat, row_splits, Wq, bq, Wk, bk, Wv, bv, Wp1, bp1, gp,
           betap, Wp2, bp2, gw1, betaw1, Ww1, bww1, gw2, betaw2, Ww2, bww2):
    table, feat_q, t_own = _project(feat, point, Wq, bq, Wk, bk, Wv, bv, Wp1)

    pts_pad = jnp.full((NPAD, 8), 1e4, jnp.float32)
    pts_pad = lax.dynamic_update_slice(
        pts_pad, jnp.pad(point, ((0, 0), (0, 5))), (0, 0))
    pts_t = pts_pad.T

    outs = []
    for h in range(2):
        q_h = lax.dynamic_slice(pts_pad, (h * NH, 0), (NPAD_H, 8))
        idx_h = _knn(q_h, pts_t)[:NH]
        idx_flat = jnp.pad(idx_h.reshape(-1), (0, ROWS_PAD - ROWS))
        gathered = _sc_gather(table, idx_flat)[:ROWS]
        out_h = _attn(gathered,
                      lax.dynamic_slice(feat_q, (h * NH, 0), (NH, D)),
                      lax.dynamic_slice(t_own, (h * NH, 0), (NH, NS)),
                      bp1, gp, betap, Wp2, bp2, gw1, betaw1,
                      Ww1, bww1, gw2, betaw2, Ww2, bww2)
        outs.append(out_h)
    return jnp.concatenate(outs, axis=0)


# speedup vs baseline: 7.9565x; 1.0345x over previous
import functools

import jax
import jax.numpy as jnp
from jax import lax
from jax.experimental import pallas as pl
from jax.experimental.pallas import tpu as pltpu
from jax.experimental.pallas import tpu_sc as plsc

N = 10000
NPAD = 10240
D = 128
NS = 16
SHARE = 8
MIDS = D // SHARE
EPS = 1e-5
TBL = 384

NH = 5000
NPAD_H = 5120

NW = 32
ROWS = NH * NS
ROWS_PAD = 81920
RPW = ROWS_PAD // NW
CHUNK = 128
NCHUNK = RPW // CHUNK

_HI = jax.lax.Precision.HIGHEST


def _k1_body(feat_ref, point_ref, wq_ref, bq_ref, wk_ref, bk_ref, wv_ref,
             bv_ref, wp1_ref, table_ref, fq_ref, tq_ref):
    f = feat_ref[...]
    p = point_ref[...]
    fq_ref[...] = jnp.dot(f, wq_ref[...], precision=_HI) + bq_ref[...]
    table_ref[:, 0:D] = jnp.dot(f, wk_ref[...], precision=_HI) + bk_ref[...]
    table_ref[:, D:2 * D] = jnp.dot(f, wv_ref[...], precision=_HI) + bv_ref[...]
    t = jnp.dot(p, wp1_ref[...], precision=_HI)
    table_ref[:, 2 * D:TBL] = jnp.concatenate(
        [t, jnp.zeros((t.shape[0], TBL - 2 * D - 3), jnp.float32)], axis=1)
    tq_ref[...] = jnp.concatenate(
        [t, jnp.zeros((t.shape[0], NS - 3), jnp.float32)], axis=1)


def _project(feat, point, Wq, bq, Wk, bk, Wv, bv, Wp1):
    B = 400
    grid = N // B
    return pl.pallas_call(
        _k1_body,
        grid=(grid,),
        in_specs=[
            pl.BlockSpec((B, D), lambda i: (i, 0)),
            pl.BlockSpec((B, 3), lambda i: (i, 0)),
            pl.BlockSpec((D, D), lambda i: (0, 0)),
            pl.BlockSpec((D,), lambda i: (0,)),
            pl.BlockSpec((D, D), lambda i: (0, 0)),
            pl.BlockSpec((D,), lambda i: (0,)),
            pl.BlockSpec((D, D), lambda i: (0, 0)),
            pl.BlockSpec((D,), lambda i: (0,)),
            pl.BlockSpec((3, 3), lambda i: (0, 0)),
        ],
        out_specs=[
            pl.BlockSpec((B, TBL), lambda i: (i, 0)),
            pl.BlockSpec((B, D), lambda i: (i, 0)),
            pl.BlockSpec((B, NS), lambda i: (i, 0)),
        ],
        out_shape=[
            jax.ShapeDtypeStruct((N, TBL), jnp.float32),
            jax.ShapeDtypeStruct((N, D), jnp.float32),
            jax.ShapeDtypeStruct((N, NS), jnp.float32),
        ],
    )(feat, point, Wq, bq, Wk, bk, Wv, bv, Wp1)


def _k2_body(q_ref, kt_ref, idx_ref):
    q = q_ref[...]
    kt = kt_ref[...]
    qk = jnp.dot((-2.0 * q).astype(jnp.bfloat16), kt.astype(jnp.bfloat16),
                 preferred_element_type=jnp.float32)
    sqk = jnp.sum(kt * kt, axis=0, keepdims=True)
    vals = sqk + qk
    bq = q.shape[0]

    W = 256
    G = NPAD // W
    BIGF = jnp.float32(3e38)
    BIGI = jnp.int32(2**30)
    m1 = jnp.full((bq, W), BIGF, jnp.float32)
    m2 = jnp.full((bq, W), BIGF, jnp.float32)
    m3 = jnp.full((bq, W), BIGF, jnp.float32)
    zi = jnp.zeros((bq, W), jnp.int32)
    a1, a2, a3 = zi, zi, zi
    for g in range(G):
        x = vals[:, g * W:(g + 1) * W]
        gi = jnp.int32(g)
        lt1 = x < m1
        lt2 = x < m2
        lt3 = x < m3
        a3 = jnp.where(lt2, a2, jnp.where(lt3, gi, a3))
        m3 = jnp.where(lt2, m2, jnp.where(lt3, x, m3))
        a2 = jnp.where(lt1, a1, jnp.where(lt2, gi, a2))
        m2 = jnp.where(lt1, m1, jnp.where(lt2, x, m2))
        a1 = jnp.where(lt1, gi, a1)
        m1 = jnp.where(lt1, x, m1)

    lane = lax.broadcasted_iota(jnp.int32, (bq, W), 1)
    cols = []
    for _ in range(NS):
        m = jnp.min(m1, axis=1, keepdims=True)
        cand = jnp.where(m1 == m, a1 * W + lane, BIGI)
        col = jnp.min(cand, axis=1, keepdims=True)
        cols.append(col)
        sel = lane == (col & (W - 1))
        m1 = jnp.where(sel, m2, m1)
        a1 = jnp.where(sel, a2, a1)
        m2 = jnp.where(sel, m3, m2)
        a2 = jnp.where(sel, a3, a2)
        m3 = jnp.where(sel, BIGF, m3)
    idx_ref[...] = jnp.concatenate(cols, axis=1)


def _knn(pts_pad_q, pts_t):
    BQ = 512
    nq = pts_pad_q.shape[0]
    grid = nq // BQ
    return pl.pallas_call(
        _k2_body,
        grid=(grid,),
        in_specs=[
            pl.BlockSpec((BQ, 8), lambda i: (i, 0)),
            pl.BlockSpec((8, NPAD), lambda i: (0, 0)),
        ],
        out_specs=pl.BlockSpec((BQ, NS), lambda i: (i, 0)),
        out_shape=jax.ShapeDtypeStruct((nq, NS), jnp.int32),
    )(pts_pad_q, pts_t)


def _sc_gather(table, idx_flat):
    mesh = plsc.VectorSubcoreMesh(core_axis_name="c", subcore_axis_name="s")

    @functools.partial(
        pl.kernel,
        mesh=mesh,
        out_type=jax.ShapeDtypeStruct((ROWS_PAD, TBL), jnp.float32),
        scratch_types=[
            pltpu.VMEM((RPW,), jnp.int32),
            pltpu.VMEM((CHUNK, TBL), jnp.float32),
            pltpu.VMEM((CHUNK, TBL), jnp.float32),
            pltpu.SemaphoreType.DMA,
            pltpu.SemaphoreType.DMA,
            pltpu.SemaphoreType.DMA,
            pltpu.SemaphoreType.DMA,
        ],
    )
    def gather_kernel(table_hbm, idx_hbm, out_hbm, idx_v, buf0, buf1,
                      sg0, sg1, ss0, ss1):
        wid = lax.axis_index("s") * 2 + lax.axis_index("c")
        base = wid * RPW
        pltpu.sync_copy(idx_hbm.at[pl.ds(base, RPW)], idx_v)

        def pair(i, carry):
            c0 = 2 * i
            c1 = 2 * i + 1
            o0 = base + c0 * CHUNK
            o1 = base + c1 * CHUNK

            @pl.when(i > 0)
            def _():
                pltpu.make_async_copy(
                    buf0, out_hbm.at[pl.ds(o0, CHUNK)], ss0).wait()

            g0 = pltpu.async_copy(
                table_hbm.at[idx_v.at[pl.ds(c0 * CHUNK, CHUNK)]], buf0, sg0)

            @pl.when(i > 0)
            def _():
                pltpu.make_async_copy(
                    buf1, out_hbm.at[pl.ds(o1, CHUNK)], ss1).wait()

            g1 = pltpu.async_copy(
                table_hbm.at[idx_v.at[pl.ds(c1 * CHUNK, CHUNK)]], buf1, sg1)
            g0.wait()
            pltpu.async_copy(buf0, out_hbm.at[pl.ds(o0, CHUNK)], ss0)
            g1.wait()
            pltpu.async_copy(buf1, out_hbm.at[pl.ds(o1, CHUNK)], ss1)
            return carry

        lax.fori_loop(0, NCHUNK // 2, pair, 0)
        pltpu.make_async_copy(buf0, out_hbm.at[pl.ds(base, CHUNK)], ss0).wait()
        pltpu.make_async_copy(buf1, out_hbm.at[pl.ds(base, CHUNK)], ss1).wait()

    return gather_kernel(table, idx_flat)


def _k4_body(g_ref, q_ref, tq_ref, bp1_ref, gp_ref, bpp_ref, wp2_ref,
             bp2_ref, gw1_ref, bw1_ref, ww1_ref, bww1_ref, gw2_ref, bw2_ref,
             ww2_ref, bww2_ref, out_ref):
    B = q_ref.shape[0]
    BL = B * NS
    inv = jnp.float32(1.0 / jnp.sqrt(1.0 + EPS))

    g = g_ref[...]
    kg = g[:, 0:D]
    vg = g[:, D:2 * D]
    tn = g[:, 2 * D:2 * D + 3]

    ti = tq_ref[:, 0:3]
    pr3 = tn.reshape(B, NS, 3) - ti[:, None, :]
    pr3 = pr3 + bp1_ref[...]
    pr3 = jax.nn.relu(pr3 * inv * gp_ref[...] + bpp_ref[...])
    point_r = jnp.dot(pr3.reshape(BL, 3), wp2_ref[...],
                      precision=_HI) + bp2_ref[...]

    q = q_ref[...]
    qb = jnp.broadcast_to(q[:, None, :], (B, NS, D)).reshape(BL, D)
    w = kg - qb + point_r
    w = jax.nn.relu(w * inv * gw1_ref[...] + bw1_ref[...])
    w = jnp.dot(w, ww1_ref[...], precision=_HI) + bww1_ref[...]
    w = jax.nn.relu(w * inv * gw2_ref[...] + bw2_ref[...])
    w = jnp.dot(w, ww2_ref[...], precision=_HI) + bww2_ref[...]

    m = jnp.max(w, axis=1, keepdims=True)
    e = jnp.exp(w - m)
    w = e / jnp.sum(e, axis=1, keepdims=True)

    wt = jnp.concatenate([w] * SHARE, axis=1)
    fv = (vg + point_r) * wt
    out_ref[...] = jnp.sum(fv.reshape(B, NS, D), axis=1)


def _attn(gathered, feat_q, t_own, bp1, gp, betap, Wp2, bp2, gw1, betaw1,
          Ww1, bww1, gw2, betaw2, Ww2, bww2):
    B = 200
    grid = NH // B
    full = lambda shape: pl.BlockSpec(shape, lambda i: tuple(0 for _ in shape))
    return pl.pallas_call(
        _k4_body,
        grid=(grid,),
        in_specs=[
            pl.BlockSpec((B * NS, TBL), lambda i: (i, 0)),
            pl.BlockSpec((B, D), lambda i: (i, 0)),
            pl.BlockSpec((B, NS), lambda i: (i, 0)),
            full((3,)), full((3,)), full((3,)),
            full((3, D)), full((D,)),
            full((D,)), full((D,)),
            full((D, MIDS)), full((MIDS,)),
            full((MIDS,)), full((MIDS,)),
            full((MIDS, MIDS)), full((MIDS,)),
        ],
        out_specs=pl.BlockSpec((B, D), lambda i: (i, 0)),
        out_shape=jax.ShapeDtypeStruct((NH, D), jnp.float32),
    )(gathered, feat_q, t_own, bp1, gp, betap, Wp2, bp2, gw1, betaw1,
      Ww1, bww1, gw2, betaw2, Ww2, bww2)


def kernel(point, feat, row_splits, Wq, bq, Wk, bk, Wv, bv, Wp1, bp1, gp,
           betap, Wp2, bp2, gw1, betaw1, Ww1, bww1, gw2, betaw2, Ww2, bww2):
    table, feat_q, t_own = _project(feat, point, Wq, bq, Wk, bk, Wv, bv, Wp1)

    pts_pad = jnp.full((NPAD, 8), 1e4, jnp.float32)
    pts_pad = lax.dynamic_update_slice(
        pts_pad, jnp.pad(point, ((0, 0), (0, 5))), (0, 0))
    pts_t = pts_pad.T

    outs = []
    for h in range(2):
        q_h = lax.dynamic_slice(pts_pad, (h * NH, 0), (NPAD_H, 8))
        idx_h = _knn(q_h, pts_t)[:NH]
        idx_flat = jnp.pad(idx_h.reshape(-1), (0, ROWS_PAD - ROWS))
        gathered = _sc_gather(table, idx_flat)[:ROWS]
        out_h = _attn(gathered,
                      lax.dynamic_slice(feat_q, (h * NH, 0), (NH, D)),
                      lax.dynamic_slice(t_own, (h * NH, 0), (NH, NS)),
                      bp1, gp, betap, Wp2, bp2, gw1, betaw1,
                      Ww1, bww1, gw2, betaw2, Ww2, bww2)
        outs.append(out_h)
    return jnp.concatenate(outs, axis=0)
